# Initial kernel scaffold; baseline (speedup 1.0000x reference)
import jax
import jax.numpy as jnp
from jax.experimental import pallas as pl


def kernel(adj_indices, adj_values, user_emb, item_emb, u_w0, v_w0, u_w1, v_w1):
    raise NotImplementedError("write your pallas kernel here")


# speedup vs baseline: 15.5475x; 15.5475x over previous
import functools

import jax
import jax.numpy as jnp
from jax import lax
from jax.experimental import pallas as pl
from jax.experimental.pallas import tpu as pltpu
from jax.experimental.pallas import tpu_sc as plsc

USER_N = 60000
ITEM_N = 40000
NN = USER_N + ITEM_N
EDGES = 3200000
H = 16
SLOPE = 0.2

NC = 2
NS = 16
NW = NC * NS
CHUNK = 128
CPW = 800
SUP = 32
NSUP = CPW // SUP
EPAD = NW * CPW * CHUNK
NPAD = 100096
RPS = NPAD // NS

_BC_DNUMS = lax.GatherDimensionNumbers(
    offset_dims=(), collapsed_slice_dims=(0,), start_index_map=(0,))


def _bcast_lane(vec, lane):
    idx = jnp.full((16, 1), lane, jnp.int32)
    return lax.gather(vec, idx, _BC_DNUMS, (1,),
                      mode=lax.GatherScatterMode.PROMISE_IN_BOUNDS)


def _spmm_sc(rows2d, cols2d, vals2d, feat, zeros):
    mesh = plsc.VectorSubcoreMesh(core_axis_name="c", subcore_axis_name="s")

    @functools.partial(
        pl.kernel,
        out_type=jax.ShapeDtypeStruct((NC, NPAD, H), jnp.float32),
        mesh=mesh,
        scratch_types=[
            pltpu.VMEM((SUP, CHUNK), jnp.int32),
            pltpu.VMEM((SUP, CHUNK), jnp.int32),
            pltpu.VMEM((SUP, CHUNK), jnp.float32),
            pltpu.VMEM((CHUNK, H), jnp.float32),
            pltpu.VMEM_SHARED((NPAD, H), jnp.float32),
            pltpu.SemaphoreType.DMA,
        ],
        compiler_params=pltpu.CompilerParams(use_tc_tiling_on_sc=False),
    )
    def k(rows_hbm, cols_hbm, vals_hbm, feat_hbm, zero_hbm, out_hbm,
          row_v, col_v, val_v, gat_v, acc, sem):
        cid = lax.axis_index("c")
        sid = lax.axis_index("s")
        wid = cid * NS + sid

        pltpu.sync_copy(zero_hbm.at[pl.ds(sid * RPS, RPS)],
                        acc.at[pl.ds(sid * RPS, RPS)])
        plsc.subcore_barrier()

        base_chunk = wid * CPW

        @pl.loop(0, NSUP)
        def _(si):
            c0 = base_chunk + si * SUP
            pltpu.sync_copy(rows_hbm.at[pl.ds(c0, SUP)], row_v)
            pltpu.sync_copy(cols_hbm.at[pl.ds(c0, SUP)], col_v)
            pltpu.sync_copy(vals_hbm.at[pl.ds(c0, SUP)], val_v)

            @pl.loop(0, SUP)
            def _(j):
                pltpu.async_copy(feat_hbm.at[col_v.at[j]], gat_v, sem).wait()

                @pl.loop(0, CHUNK // 16)
                def _(g):
                    vv = val_v[j, pl.ds(g * 16, 16)]
                    for l in range(16):
                        bc = _bcast_lane(vv, l)
                        gat_v[g * 16 + l, :] = gat_v[g * 16 + l, :] * bc

                pltpu.sync_copy(gat_v, acc.at[row_v.at[j]], add=True)

        plsc.subcore_barrier()
        pltpu.sync_copy(acc.at[pl.ds(sid * RPS, RPS)],
                        out_hbm.at[cid, pl.ds(sid * RPS, RPS)])

    return k(rows2d, cols2d, vals2d, feat, zeros)


BLK = 4000
UBLK = USER_N // BLK
NBLK = NN // BLK


def _transform_tc(emb, uw, vw):
    def body(emb_ref, uw_ref, vw_ref, out_ref):
        i = pl.program_id(0)
        w = jnp.where(i < UBLK, uw_ref[...], vw_ref[...])
        out_ref[...] = jnp.dot(emb_ref[...], w,
                               preferred_element_type=jnp.float32)

    return pl.pallas_call(
        body,
        grid=(NBLK,),
        in_specs=[pl.BlockSpec((BLK, H), lambda i: (i, 0)),
                  pl.BlockSpec((H, H), lambda i: (0, 0)),
                  pl.BlockSpec((H, H), lambda i: (0, 0))],
        out_specs=pl.BlockSpec((BLK, H), lambda i: (i, 0)),
        out_shape=jax.ShapeDtypeStruct((NN, H), jnp.float32),
    )(emb, uw, vw)


def _combine_tc(p, uw=None, vw=None):
    want_feat = uw is not None

    def body(*refs):
        if want_feat:
            p_ref, uw_ref, vw_ref, nrm_ref, feat_ref = refs
        else:
            p_ref, nrm_ref = refs
        s = p_ref[0] + p_ref[1]
        emb = jnp.where(s >= 0, s, SLOPE * s)
        n2 = jnp.sum(emb * emb, axis=1, keepdims=True)
        nrm_ref[...] = emb / jnp.maximum(jnp.sqrt(n2), 1e-12)
        if want_feat:
            i = pl.program_id(0)
            w = jnp.where(i < UBLK, uw_ref[...], vw_ref[...])
            feat_ref[...] = jnp.dot(emb, w, preferred_element_type=jnp.float32)

    in_specs = [pl.BlockSpec((NC, BLK, H), lambda i: (0, i, 0))]
    operands = [p]
    out_shape = [jax.ShapeDtypeStruct((NN, H), jnp.float32)]
    out_specs = [pl.BlockSpec((BLK, H), lambda i: (i, 0))]
    if want_feat:
        in_specs += [pl.BlockSpec((H, H), lambda i: (0, 0)),
                     pl.BlockSpec((H, H), lambda i: (0, 0))]
        operands += [uw, vw]
        out_shape.append(jax.ShapeDtypeStruct((NN, H), jnp.float32))
        out_specs.append(pl.BlockSpec((BLK, H), lambda i: (i, 0)))

    return pl.pallas_call(
        body,
        grid=(NBLK,),
        in_specs=in_specs,
        out_specs=out_specs,
        out_shape=out_shape,
    )(*operands)


def kernel(adj_indices, adj_values, user_emb, item_emb, u_w0, v_w0, u_w1, v_w1):
    pad_e = EPAD - EDGES
    ipad = jnp.zeros((pad_e,), adj_indices.dtype)
    rows2d = jnp.concatenate([adj_indices[0], ipad]).reshape(EPAD // CHUNK, CHUNK)
    cols2d = jnp.concatenate([adj_indices[1], ipad]).reshape(EPAD // CHUNK, CHUNK)
    vals2d = jnp.concatenate(
        [adj_values, jnp.zeros((pad_e,), jnp.float32)]).reshape(EPAD // CHUNK, CHUNK)
    zeros = jnp.zeros((NPAD, H), jnp.float32)

    emb0 = jnp.concatenate([user_emb, item_emb], axis=0)
    feat0 = _transform_tc(emb0, u_w0, v_w0)

    p1 = _spmm_sc(rows2d, cols2d, vals2d, feat0, zeros)
    nrm1, feat1 = _combine_tc(p1, u_w1, v_w1)

    p2 = _spmm_sc(rows2d, cols2d, vals2d, feat1, zeros)
    (nrm2,) = _combine_tc(p2)

    user_embedding = jnp.concatenate(
        [user_emb, nrm1[:USER_N], nrm2[:USER_N]], axis=1)
    item_embedding = jnp.concatenate(
        [item_emb, nrm1[USER_N:], nrm2[USER_N:]], axis=1)
    return user_embedding, item_embedding


# speedup vs baseline: 21.7290x; 1.3976x over previous
import functools

import jax
import jax.numpy as jnp
from jax import lax
from jax.experimental import pallas as pl
from jax.experimental.pallas import tpu as pltpu
from jax.experimental.pallas import tpu_sc as plsc

USER_N = 60000
ITEM_N = 40000
NN = USER_N + ITEM_N
EDGES = 3200000
H = 16
SLOPE = 0.2

NC = 2
NS = 16
NW = NC * NS
CHUNK = 128
CPW = 800
SUP = 16
NSUP = CPW // SUP
NBUF = 8
ROUNDS = SUP // NBUF
EPAD = NW * CPW * CHUNK
NPAD = 100096
RPS = NPAD // NS

_BC_DNUMS = lax.GatherDimensionNumbers(
    offset_dims=(), collapsed_slice_dims=(0,), start_index_map=(0,))


def _bcast_lane(vec, lane):
    idx = jnp.full((16, 1), lane, jnp.int32)
    return lax.gather(vec, idx, _BC_DNUMS, (1,),
                      mode=lax.GatherScatterMode.PROMISE_IN_BOUNDS)


def _spmm_sc(rows2d, cols2d, vals2d, feat, zeros):
    mesh = plsc.VectorSubcoreMesh(core_axis_name="c", subcore_axis_name="s")

    @functools.partial(
        pl.kernel,
        out_type=jax.ShapeDtypeStruct((NC, NPAD, H), jnp.float32),
        mesh=mesh,
        scratch_types=[
            pltpu.VMEM((SUP, CHUNK), jnp.int32),
            pltpu.VMEM((SUP, CHUNK), jnp.int32),
            pltpu.VMEM((SUP, CHUNK), jnp.float32),
            pltpu.VMEM((NBUF, CHUNK, H), jnp.float32),
            pltpu.VMEM_SHARED((NPAD, H), jnp.float32),
            pltpu.SemaphoreType.DMA((NBUF,)),
            pltpu.SemaphoreType.DMA((NBUF,)),
        ],
        compiler_params=pltpu.CompilerParams(use_tc_tiling_on_sc=False),
    )
    def k(rows_hbm, cols_hbm, vals_hbm, feat_hbm, zero_hbm, out_hbm,
          row_v, col_v, val_v, gat_v, acc, gsem, ssem):
        cid = lax.axis_index("c")
        sid = lax.axis_index("s")
        wid = cid * NS + sid

        pltpu.sync_copy(zero_hbm.at[pl.ds(sid * RPS, RPS)],
                        acc.at[pl.ds(sid * RPS, RPS)])
        plsc.subcore_barrier()

        base_chunk = wid * CPW

        @pl.loop(0, NSUP)
        def _(si):
            c0 = base_chunk + si * SUP
            pltpu.sync_copy(rows_hbm.at[pl.ds(c0, SUP)], row_v)
            pltpu.sync_copy(cols_hbm.at[pl.ds(c0, SUP)], col_v)
            pltpu.sync_copy(vals_hbm.at[pl.ds(c0, SUP)], val_v)

            for r in range(ROUNDS):
                gets = [
                    pltpu.async_copy(feat_hbm.at[col_v.at[r * NBUF + u]],
                                     gat_v.at[u], gsem.at[u])
                    for u in range(NBUF)
                ]
                puts = []
                for u in range(NBUF):
                    j = r * NBUF + u
                    gets[u].wait()

                    @pl.loop(0, CHUNK // 16)
                    def _(g, u=u, j=j):
                        vv = val_v[j, pl.ds(g * 16, 16)]
                        for l in range(16):
                            bc = _bcast_lane(vv, l)
                            gat_v[u, g * 16 + l, :] = (
                                gat_v[u, g * 16 + l, :] * bc)

                    puts.append(
                        pltpu.async_copy(gat_v.at[u], acc.at[row_v.at[j]],
                                         ssem.at[u], add=True))
                for p in puts:
                    p.wait()

        plsc.subcore_barrier()
        pltpu.sync_copy(acc.at[pl.ds(sid * RPS, RPS)],
                        out_hbm.at[cid, pl.ds(sid * RPS, RPS)])

    return k(rows2d, cols2d, vals2d, feat, zeros)


BLK = 4000
UBLK = USER_N // BLK
NBLK = NN // BLK


def _transform_tc(emb, uw, vw):
    def body(emb_ref, uw_ref, vw_ref, out_ref):
        i = pl.program_id(0)
        w = jnp.where(i < UBLK, uw_ref[...], vw_ref[...])
        out_ref[...] = jnp.dot(emb_ref[...], w,
                               preferred_element_type=jnp.float32)

    return pl.pallas_call(
        body,
        grid=(NBLK,),
        in_specs=[pl.BlockSpec((BLK, H), lambda i: (i, 0)),
                  pl.BlockSpec((H, H), lambda i: (0, 0)),
                  pl.BlockSpec((H, H), lambda i: (0, 0))],
        out_specs=pl.BlockSpec((BLK, H), lambda i: (i, 0)),
        out_shape=jax.ShapeDtypeStruct((NN, H), jnp.float32),
    )(emb, uw, vw)


def _combine_tc(p, uw=None, vw=None):
    want_feat = uw is not None

    def body(*refs):
        if want_feat:
            p_ref, uw_ref, vw_ref, nrm_ref, feat_ref = refs
        else:
            p_ref, nrm_ref = refs
        s = p_ref[0] + p_ref[1]
        emb = jnp.where(s >= 0, s, SLOPE * s)
        n2 = jnp.sum(emb * emb, axis=1, keepdims=True)
        nrm_ref[...] = emb / jnp.maximum(jnp.sqrt(n2), 1e-12)
        if want_feat:
            i = pl.program_id(0)
            w = jnp.where(i < UBLK, uw_ref[...], vw_ref[...])
            feat_ref[...] = jnp.dot(emb, w, preferred_element_type=jnp.float32)

    in_specs = [pl.BlockSpec((NC, BLK, H), lambda i: (0, i, 0))]
    operands = [p]
    out_shape = [jax.ShapeDtypeStruct((NN, H), jnp.float32)]
    out_specs = [pl.BlockSpec((BLK, H), lambda i: (i, 0))]
    if want_feat:
        in_specs += [pl.BlockSpec((H, H), lambda i: (0, 0)),
                     pl.BlockSpec((H, H), lambda i: (0, 0))]
        operands += [uw, vw]
        out_shape.append(jax.ShapeDtypeStruct((NN, H), jnp.float32))
        out_specs.append(pl.BlockSpec((BLK, H), lambda i: (i, 0)))

    return pl.pallas_call(
        body,
        grid=(NBLK,),
        in_specs=in_specs,
        out_specs=out_specs,
        out_shape=out_shape,
    )(*operands)


def kernel(adj_indices, adj_values, user_emb, item_emb, u_w0, v_w0, u_w1, v_w1):
    pad_e = EPAD - EDGES
    ipad = jnp.zeros((pad_e,), adj_indices.dtype)
    rows2d = jnp.concatenate([adj_indices[0], ipad]).reshape(EPAD // CHUNK, CHUNK)
    cols2d = jnp.concatenate([adj_indices[1], ipad]).reshape(EPAD // CHUNK, CHUNK)
    vals2d = jnp.concatenate(
        [adj_values, jnp.zeros((pad_e,), jnp.float32)]).reshape(EPAD // CHUNK, CHUNK)
    zeros = jnp.zeros((NPAD, H), jnp.float32)

    emb0 = jnp.concatenate([user_emb, item_emb], axis=0)
    feat0 = _transform_tc(emb0, u_w0, v_w0)

    p1 = _spmm_sc(rows2d, cols2d, vals2d, feat0, zeros)
    nrm1, feat1 = _combine_tc(p1, u_w1, v_w1)

    p2 = _spmm_sc(rows2d, cols2d, vals2d, feat1, zeros)
    (nrm2,) = _combine_tc(p2)

    user_embedding = jnp.concatenate(
        [user_emb, nrm1[:USER_N], nrm2[:USER_N]], axis=1)
    item_embedding = jnp.concatenate(
        [item_emb, nrm1[USER_N:], nrm2[USER_N:]], axis=1)
    return user_embedding, item_embedding


# speedup vs baseline: 39.6376x; 1.8242x over previous
import functools

import jax
import jax.numpy as jnp
from jax import lax
from jax.experimental import pallas as pl
from jax.experimental.pallas import tpu as pltpu
from jax.experimental.pallas import tpu_sc as plsc

USER_N = 60000
ITEM_N = 40000
NN = USER_N + ITEM_N
EDGES = 3200000
H = 16
SLOPE = 0.2

NC = 2
NS = 16
NW = NC * NS
CHUNK = 128
NCHUNKS = EDGES // CHUNK
BASE_CPW = NCHUNKS // NW
EXTRA = NCHUNKS - NW * BASE_CPW
SUP = 16
FULL_STEPS = BASE_CPW // SUP
TAIL = BASE_CPW - FULL_STEPS * SUP
NBUF = 8
NPAD = 100096
RPS = NPAD // NS
GROWS = NBUF * CHUNK

_BC_DNUMS = lax.GatherDimensionNumbers(
    offset_dims=(), collapsed_slice_dims=(0,), start_index_map=(0,))


def _bcast_lane(vec, lane):
    idx = jnp.full((16, 1), lane, jnp.int32)
    return lax.gather(vec, idx, _BC_DNUMS, (1,),
                      mode=lax.GatherScatterMode.PROMISE_IN_BOUNDS)


def _spmm_sc(rows2d, cols2d, vals2d, feat):
    mesh = plsc.VectorSubcoreMesh(core_axis_name="c", subcore_axis_name="s")

    @functools.partial(
        pl.kernel,
        out_type=jax.ShapeDtypeStruct((NC, NPAD, H), jnp.float32),
        mesh=mesh,
        scratch_types=[
            pltpu.VMEM((SUP, CHUNK), jnp.int32),
            pltpu.VMEM((SUP, CHUNK), jnp.int32),
            pltpu.VMEM((SUP, CHUNK), jnp.float32),
            pltpu.VMEM((GROWS, H), jnp.float32),
            pltpu.VMEM_SHARED((NPAD, H), jnp.float32),
            pltpu.SemaphoreType.DMA((NBUF,)),
            pltpu.SemaphoreType.DMA((NBUF,)),
        ],
        compiler_params=pltpu.CompilerParams(use_tc_tiling_on_sc=False),
    )
    def k(rows_hbm, cols_hbm, vals_hbm, feat_hbm, out_hbm,
          row_v, col_v, val_v, gat_v, acc, gsem, ssem):
        cid = lax.axis_index("c")
        sid = lax.axis_index("s")
        wid = cid * NS + sid

        @pl.loop(0, GROWS)
        def _(g):
            gat_v[g, :] = jnp.zeros((16,), jnp.float32)

        for t in range(RPS // GROWS):
            pltpu.sync_copy(gat_v, acc.at[pl.ds(sid * RPS + t * GROWS, GROWS)])
        rem = RPS % GROWS
        if rem:
            pltpu.sync_copy(
                gat_v.at[pl.ds(0, rem)],
                acc.at[pl.ds(sid * RPS + (RPS // GROWS) * GROWS, rem)])
        plsc.subcore_barrier()

        def process_staged(count):
            for r0 in range(0, count, NBUF):
                n = min(NBUF, count - r0)
                gets = [
                    pltpu.async_copy(feat_hbm.at[col_v.at[r0 + u]],
                                     gat_v.at[pl.ds(u * CHUNK, CHUNK)],
                                     gsem.at[u])
                    for u in range(n)
                ]
                puts = []
                for u in range(n):
                    j = r0 + u
                    gets[u].wait()

                    @pl.loop(0, CHUNK // 16)
                    def _(g, u=u, j=j):
                        vv = val_v[j, pl.ds(g * 16, 16)]
                        for l in range(16):
                            bc = _bcast_lane(vv, l)
                            kk = u * CHUNK + g * 16 + l
                            gat_v[kk, :] = gat_v[kk, :] * bc

                    puts.append(
                        pltpu.async_copy(gat_v.at[pl.ds(u * CHUNK, CHUNK)],
                                         acc.at[row_v.at[j]],
                                         ssem.at[u], add=True))
                for p in puts:
                    p.wait()

        lo = wid * BASE_CPW + jnp.minimum(wid, EXTRA)

        @pl.loop(0, FULL_STEPS)
        def _(si):
            c0 = lo + si * SUP
            pltpu.sync_copy(rows_hbm.at[pl.ds(c0, SUP)], row_v)
            pltpu.sync_copy(cols_hbm.at[pl.ds(c0, SUP)], col_v)
            pltpu.sync_copy(vals_hbm.at[pl.ds(c0, SUP)], val_v)
            process_staged(SUP)

        ct = lo + FULL_STEPS * SUP
        pltpu.sync_copy(rows_hbm.at[pl.ds(ct, TAIL)], row_v.at[pl.ds(0, TAIL)])
        pltpu.sync_copy(cols_hbm.at[pl.ds(ct, TAIL)], col_v.at[pl.ds(0, TAIL)])
        pltpu.sync_copy(vals_hbm.at[pl.ds(ct, TAIL)], val_v.at[pl.ds(0, TAIL)])
        process_staged(TAIL)

        @pl.when(wid < EXTRA)
        def _():
            ce = NW * BASE_CPW + wid
            pltpu.sync_copy(rows_hbm.at[pl.ds(ce, 1)], row_v.at[pl.ds(0, 1)])
            pltpu.sync_copy(cols_hbm.at[pl.ds(ce, 1)], col_v.at[pl.ds(0, 1)])
            pltpu.sync_copy(vals_hbm.at[pl.ds(ce, 1)], val_v.at[pl.ds(0, 1)])
            process_staged(1)

        plsc.subcore_barrier()
        pltpu.sync_copy(acc.at[pl.ds(sid * RPS, RPS)],
                        out_hbm.at[cid, pl.ds(sid * RPS, RPS)])

    return k(rows2d, cols2d, vals2d, feat)


BLK = 4000
UBLK = USER_N // BLK
NBLK = NN // BLK


def _transform_tc(emb, uw, vw):
    def body(emb_ref, uw_ref, vw_ref, out_ref):
        i = pl.program_id(0)
        w = jnp.where(i < UBLK, uw_ref[...], vw_ref[...])
        out_ref[...] = jnp.dot(emb_ref[...], w,
                               preferred_element_type=jnp.float32)

    return pl.pallas_call(
        body,
        grid=(NBLK,),
        in_specs=[pl.BlockSpec((BLK, H), lambda i: (i, 0)),
                  pl.BlockSpec((H, H), lambda i: (0, 0)),
                  pl.BlockSpec((H, H), lambda i: (0, 0))],
        out_specs=pl.BlockSpec((BLK, H), lambda i: (i, 0)),
        out_shape=jax.ShapeDtypeStruct((NN, H), jnp.float32),
    )(emb, uw, vw)


def _combine_tc(p, uw=None, vw=None):
    want_feat = uw is not None

    def body(*refs):
        if want_feat:
            p_ref, uw_ref, vw_ref, nrm_ref, feat_ref = refs
        else:
            p_ref, nrm_ref = refs
        s = p_ref[0] + p_ref[1]
        emb = jnp.where(s >= 0, s, SLOPE * s)
        n2 = jnp.sum(emb * emb, axis=1, keepdims=True)
        nrm_ref[...] = emb / jnp.maximum(jnp.sqrt(n2), 1e-12)
        if want_feat:
            i = pl.program_id(0)
            w = jnp.where(i < UBLK, uw_ref[...], vw_ref[...])
            feat_ref[...] = jnp.dot(emb, w, preferred_element_type=jnp.float32)

    in_specs = [pl.BlockSpec((NC, BLK, H), lambda i: (0, i, 0))]
    operands = [p]
    out_shape = [jax.ShapeDtypeStruct((NN, H), jnp.float32)]
    out_specs = [pl.BlockSpec((BLK, H), lambda i: (i, 0))]
    if want_feat:
        in_specs += [pl.BlockSpec((H, H), lambda i: (0, 0)),
                     pl.BlockSpec((H, H), lambda i: (0, 0))]
        operands += [uw, vw]
        out_shape.append(jax.ShapeDtypeStruct((NN, H), jnp.float32))
        out_specs.append(pl.BlockSpec((BLK, H), lambda i: (i, 0)))

    return pl.pallas_call(
        body,
        grid=(NBLK,),
        in_specs=in_specs,
        out_specs=out_specs,
        out_shape=out_shape,
    )(*operands)


def kernel(adj_indices, adj_values, user_emb, item_emb, u_w0, v_w0, u_w1, v_w1):
    rows2d = adj_indices[0].reshape(NCHUNKS, CHUNK)
    cols2d = adj_indices[1].reshape(NCHUNKS, CHUNK)
    vals2d = adj_values.reshape(NCHUNKS, CHUNK)

    emb0 = jnp.concatenate([user_emb, item_emb], axis=0)
    feat0 = _transform_tc(emb0, u_w0, v_w0)

    p1 = _spmm_sc(rows2d, cols2d, vals2d, feat0)
    nrm1, feat1 = _combine_tc(p1, u_w1, v_w1)

    p2 = _spmm_sc(rows2d, cols2d, vals2d, feat1)
    (nrm2,) = _combine_tc(p2)

    user_embedding = jnp.concatenate(
        [user_emb, nrm1[:USER_N], nrm2[:USER_N]], axis=1)
    item_embedding = jnp.concatenate(
        [item_emb, nrm1[USER_N:], nrm2[USER_N:]], axis=1)
    return user_embedding, item_embedding
